# baseline (speedup 1.0000x reference)
import jax
import jax.numpy as jnp
from jax.experimental import pallas as pl
from jax.experimental.pallas import tpu as pltpu

_N = 8
_D = 32
_H = 32
_T = 4
_BL = 16384
_HO = 16


def _score_kernel(xt_ref, w1t_ref, b1bc_ref, wh_ref, out_ref):
    w1t = w1t_ref[:, 0:_D]
    b1bc = b1bc_ref[...]
    acc = jnp.zeros((_H, _BL), jnp.float32)
    for n in range(_N):
        hn = jax.lax.dot_general(
            w1t, xt_ref[n], (((1,), (0,)), ((), ())),
            preferred_element_type=jnp.float32)
        acc = acc + jnp.maximum(hn + b1bc, 0.0)
    out_ref[...] = jax.lax.dot_general(
        wh_ref[...], acc, (((0,), (0,)), ((), ())),
        preferred_element_type=jnp.float32)


@jax.jit
def _forward(x, w1_aug, w_heads):
    b = x.shape[0]
    n_blocks = pl.cdiv(b, _BL)
    b_pad = n_blocks * _BL

    xt = jnp.transpose(x, (1, 2, 0))
    if b_pad != b:
        xt = jnp.pad(xt, ((0, 0), (0, 0), (0, b_pad - b)))

    w1t = w1_aug.T
    b1bc = jnp.broadcast_to(w1t[:, _D:_D + 1], (_H, _BL))

    rows = b_pad * _N
    flops = 2 * rows * _D * _H + 2 * b_pad * _H * _HO
    bytes_accessed = 4 * (rows * _D + (_D + 1) * _H + _H * _HO + b_pad * _HO)

    out_t = pl.pallas_call(
        _score_kernel,
        out_shape=jax.ShapeDtypeStruct((_HO, b_pad), jnp.float32),
        grid=(n_blocks,),
        in_specs=[
            pl.BlockSpec((_N, _D, _BL), lambda i: (0, 0, i)),
            pl.BlockSpec((_H, _D + 1), lambda i: (0, 0)),
            pl.BlockSpec((_H, _BL), lambda i: (0, 0)),
            pl.BlockSpec((_H, _HO), lambda i: (0, 0)),
        ],
        out_specs=pl.BlockSpec((_HO, _BL), lambda i: (0, i)),
        compiler_params=pltpu.CompilerParams(dimension_semantics=("parallel",)),
        cost_estimate=pl.CostEstimate(flops=flops, transcendentals=0,
                                      bytes_accessed=bytes_accessed),
    )(xt, w1t, b1bc, w_heads[:, :_HO])

    if b_pad != b:
        out_t = out_t[:, :b]
    return {
        "tr_pred": out_t[0:3].T,
        "rot_pred": out_t[3:6].T,
        "tor_pred": out_t[6:6 + _T].T,
    }


def kernel(x, w1_aug, w_heads):
    return _forward(x, w1_aug, w_heads)


# speedup vs baseline: 1.0293x; 1.0293x over previous
import jax
import jax.numpy as jnp
from jax.experimental import pallas as pl
from jax.experimental.pallas import tpu as pltpu

_N = 8
_D = 32
_H = 32
_T = 4
_BL = 8192
_HO = 16


def _score_kernel(xt_ref, w1t_ref, b1bc_ref, wh_ref, out_ref):
    w1t = w1t_ref[:, 0:_D]
    b1bc = b1bc_ref[...]
    acc = jnp.zeros((_H, _BL), jnp.float32)
    for n in range(_N):
        hn = jax.lax.dot_general(
            w1t, xt_ref[n], (((1,), (0,)), ((), ())),
            preferred_element_type=jnp.float32)
        acc = acc + jnp.maximum(hn + b1bc, 0.0)
    out_ref[...] = jax.lax.dot_general(
        wh_ref[...], acc, (((0,), (0,)), ((), ())),
        preferred_element_type=jnp.float32)


@jax.jit
def _forward(x, w1_aug, w_heads):
    b = x.shape[0]
    n_blocks = pl.cdiv(b, _BL)
    b_pad = n_blocks * _BL

    xt = jnp.transpose(x, (1, 2, 0))
    if b_pad != b:
        xt = jnp.pad(xt, ((0, 0), (0, 0), (0, b_pad - b)))

    w1t = w1_aug.T
    b1bc = jnp.broadcast_to(w1t[:, _D:_D + 1], (_H, _BL))

    rows = b_pad * _N
    flops = 2 * rows * _D * _H + 2 * b_pad * _H * _HO
    bytes_accessed = 4 * (rows * _D + (_D + 1) * _H + _H * _HO + b_pad * _HO)

    out_t = pl.pallas_call(
        _score_kernel,
        out_shape=jax.ShapeDtypeStruct((_HO, b_pad), jnp.float32),
        grid=(n_blocks,),
        in_specs=[
            pl.BlockSpec((_N, _D, _BL), lambda i: (0, 0, i)),
            pl.BlockSpec((_H, _D + 1), lambda i: (0, 0)),
            pl.BlockSpec((_H, _BL), lambda i: (0, 0)),
            pl.BlockSpec((_H, _HO), lambda i: (0, 0)),
        ],
        out_specs=pl.BlockSpec((_HO, _BL), lambda i: (0, i)),
        compiler_params=pltpu.CompilerParams(dimension_semantics=("parallel",)),
        cost_estimate=pl.CostEstimate(flops=flops, transcendentals=0,
                                      bytes_accessed=bytes_accessed),
    )(xt, w1t, b1bc, w_heads[:, :_HO])

    if b_pad != b:
        out_t = out_t[:, :b]
    return {
        "tr_pred": out_t[0:3].T,
        "rot_pred": out_t[3:6].T,
        "tor_pred": out_t[6:6 + _T].T,
    }


def kernel(x, w1_aug, w_heads):
    return _forward(x, w1_aug, w_heads)


# speedup vs baseline: 1.1216x; 1.0897x over previous
import jax
import jax.numpy as jnp
from jax.experimental import pallas as pl
from jax.experimental.pallas import tpu as pltpu

_N = 8
_D = 32
_H = 32
_T = 4
_BL = 8192
_HO = 16


def _score_kernel(xt_ref, w1t_ref, wh_ref, out_ref):
    w1t = w1t_ref[:, 0:_D]
    b1bc = jnp.broadcast_to(w1t_ref[:, _D:_D + 1], (_H, _BL))
    acc = jnp.zeros((_H, _BL), jnp.float32)
    for n in range(_N):
        hn = jax.lax.dot_general(
            w1t, xt_ref[n], (((1,), (0,)), ((), ())),
            preferred_element_type=jnp.float32)
        acc = acc + jnp.maximum(hn + b1bc, 0.0)
    out_ref[...] = jax.lax.dot_general(
        wh_ref[...], acc, (((0,), (0,)), ((), ())),
        preferred_element_type=jnp.float32)


@jax.jit
def _forward(x, w1_aug, w_heads):
    b = x.shape[0]
    n_blocks = pl.cdiv(b, _BL)
    b_pad = n_blocks * _BL

    xt = jnp.transpose(x, (1, 2, 0))
    if b_pad != b:
        xt = jnp.pad(xt, ((0, 0), (0, 0), (0, b_pad - b)))

    w1t = w1_aug.T

    rows = b_pad * _N
    flops = 2 * rows * _D * _H + 2 * b_pad * _H * _HO
    bytes_accessed = 4 * (rows * _D + (_D + 1) * _H + _H * _HO + b_pad * _HO)

    out_t = pl.pallas_call(
        _score_kernel,
        out_shape=jax.ShapeDtypeStruct((_HO, b_pad), jnp.float32),
        grid=(n_blocks,),
        in_specs=[
            pl.BlockSpec((_N, _D, _BL), lambda i: (0, 0, i)),
            pl.BlockSpec((_H, _D + 1), lambda i: (0, 0)),
            pl.BlockSpec((_H, _HO), lambda i: (0, 0)),
        ],
        out_specs=pl.BlockSpec((_HO, _BL), lambda i: (0, i)),
        compiler_params=pltpu.CompilerParams(dimension_semantics=("parallel",)),
        cost_estimate=pl.CostEstimate(flops=flops, transcendentals=0,
                                      bytes_accessed=bytes_accessed),
    )(xt, w1t, w_heads[:, :_HO])

    if b_pad != b:
        out_t = out_t[:, :b]
    return {
        "tr_pred": out_t[0:3].T,
        "rot_pred": out_t[3:6].T,
        "tor_pred": out_t[6:6 + _T].T,
    }


def kernel(x, w1_aug, w_heads):
    return _forward(x, w1_aug, w_heads)
